# baseline (speedup 1.0000x reference)
import functools

import jax
import jax.numpy as jnp
from jax import lax
from jax.experimental import pallas as pl
from jax.experimental.pallas import tpu as pltpu
from jax.experimental.pallas import tpu_sc as plsc

B = 1
T = 2048
C = 768
F = 1536
E = 64
K = 2
CAP = 128
N = B * T

NC = 2
NS = 16
NW = NC * NS
TPW = N // NW
TB = 512
NB = N // TB
DISP_ROWS = E * CAP
TRASH0 = DISP_ROWS + TPW
EPB = 1
DISP_PAD = DISP_ROWS + 2 * TPW


def _router_body(x_ref, wr_ref, dw1_ref, dw2_ref, wv1_ref, wv2_ref, cnt_ref):
    i = pl.program_id(0)

    @pl.when(i == 0)
    def _init():
        cnt_ref[...] = jnp.zeros_like(cnt_ref)

    x = x_ref[...]
    logits = jnp.dot(x, wr_ref[...], preferred_element_type=jnp.float32)
    m = jnp.max(logits, axis=-1, keepdims=True)
    p = jnp.exp(logits - m)
    p = p / jnp.sum(p, axis=-1, keepdims=True)

    cols = lax.broadcasted_iota(jnp.int32, (TB, E), 1)
    p1 = jnp.max(p, axis=-1, keepdims=True)
    e1 = jnp.min(jnp.where(p == p1, cols, E), axis=-1, keepdims=True)
    pm = jnp.where(cols == e1, -1.0, p)
    p2 = jnp.max(pm, axis=-1, keepdims=True)
    e2 = jnp.min(jnp.where(pm == p2, cols, E), axis=-1, keepdims=True)

    denom = p1 + p2 + 1e-9
    w1 = p1 / denom
    w2 = p2 / denom

    oh1 = (cols == e1).astype(jnp.float32)
    oh2 = (cols == e2).astype(jnp.float32)
    oh = oh1 + oh2
    r = lax.broadcasted_iota(jnp.int32, (TB, TB), 0)
    ccol = lax.broadcasted_iota(jnp.int32, (TB, TB), 1)
    tril = (r > ccol).astype(jnp.float32)
    carry = cnt_ref[0:1, :]
    cnt_excl = carry + jnp.dot(tril, oh, preferred_element_type=jnp.float32)
    cnt_ref[0:1, :] = carry + jnp.sum(oh, axis=0, keepdims=True)

    pos1 = jnp.sum(cnt_excl * oh1, axis=-1, keepdims=True).astype(jnp.int32)
    pos2 = jnp.sum(cnt_excl * oh2, axis=-1, keepdims=True).astype(jnp.int32)
    v1 = pos1 < CAP
    v2 = pos2 < CAP
    tok = lax.broadcasted_iota(jnp.int32, (TB, 1), 0)
    trash = TRASH0 + (tok % TPW)
    d1 = jnp.where(v1, e1 * CAP + pos1, trash)
    d2 = jnp.where(v2, e2 * CAP + pos2, trash)
    wv1 = jnp.where(v1, w1, 0.0)
    wv2 = jnp.where(v2, w2, 0.0)

    dw1_ref[...] = d1.reshape(1, 1, TB)
    dw2_ref[...] = d2.reshape(1, 1, TB)
    wv1_ref[...] = jnp.broadcast_to(wv1, (TB, 16)).reshape(1, TB, 16)
    wv2_ref[...] = jnp.broadcast_to(wv2, (TB, 16)).reshape(1, TB, 16)


def _run_router(xf, Wr, interpret=False):
    out3 = (
        jax.ShapeDtypeStruct((NB, 1, TB), jnp.int32),
        jax.ShapeDtypeStruct((NB, 1, TB), jnp.int32),
        jax.ShapeDtypeStruct((NB, TB, 16), jnp.float32),
        jax.ShapeDtypeStruct((NB, TB, 16), jnp.float32),
    )
    blk3 = pl.BlockSpec((1, 1, TB), lambda i: (i, 0, 0))
    blkw = pl.BlockSpec((1, TB, 16), lambda i: (i, 0, 0))
    dw1, dw2, wv1, wv2 = pl.pallas_call(
        _router_body,
        grid=(NB,),
        in_specs=[
            pl.BlockSpec((TB, C), lambda i: (i, 0)),
            pl.BlockSpec((C, E), lambda i: (0, 0)),
        ],
        out_specs=(blk3, blk3, blkw, blkw),
        out_shape=out3,
        scratch_shapes=[pltpu.VMEM((8, E), jnp.float32)],
        interpret=interpret,
    )(xf, Wr)
    return (dw1.reshape(N), dw2.reshape(N),
            wv1.reshape(N, 16), wv2.reshape(N, 16))


def _dispatch_body(x_hbm, dw1_hbm, dw2_hbm, disp_hbm, i1_v, i2_v, rows_v,
                   s1, s2):
    wid = lax.axis_index("s") * NC + lax.axis_index("c")
    base = wid * TPW
    pltpu.sync_copy(dw1_hbm.at[pl.ds(base, TPW)], i1_v)
    pltpu.sync_copy(dw2_hbm.at[pl.ds(base, TPW)], i2_v)
    pltpu.sync_copy(x_hbm.at[pl.ds(base, TPW)], rows_v)
    cp1 = pltpu.async_copy(rows_v, disp_hbm.at[i1_v], s1)
    cp2 = pltpu.async_copy(rows_v, disp_hbm.at[i2_v], s2)
    cp1.wait()
    cp2.wait()


def _sc_mesh():
    return plsc.VectorSubcoreMesh(core_axis_name="c", subcore_axis_name="s",
                                  num_cores=NC, num_subcores=NS)


def _run_dispatch(xf, dw1, dw2, interpret=False):
    mesh = _sc_mesh()
    return pl.kernel(
        _dispatch_body,
        out_type=jax.ShapeDtypeStruct((DISP_PAD, C), jnp.float32),
        mesh=mesh,
        scratch_types=[
            pltpu.VMEM((TPW,), jnp.int32),
            pltpu.VMEM((TPW,), jnp.int32),
            pltpu.VMEM((TPW, C), jnp.float32),
            pltpu.SemaphoreType.DMA,
            pltpu.SemaphoreType.DMA,
        ],
        interpret=interpret,
    )(xf, dw1, dw2)


EH = E // 2
HALF = EH * CAP
OBH = HALF + CAP


def _ffn_half_body(x_ref, w1_ref, b1_ref, w2_ref, b2_ref, o_ref):
    e = pl.program_id(0)

    @pl.when(e < EH)
    def _():
        x = x_ref[...]
        h = jnp.dot(x, w1_ref[0], preferred_element_type=jnp.float32)
        h = jnp.maximum(h + b1_ref[0], 0.0)
        o = jnp.dot(h, w2_ref[0], preferred_element_type=jnp.float32)
        o_ref[...] = o + b2_ref[0]

    @pl.when(e == EH)
    def _():
        o_ref[...] = jnp.zeros((CAP, C), jnp.float32)


def _run_ffn_half(disp, W1, b1, W2, b2, lo, interpret=False):
    wmap = lambda e: (jnp.minimum(e, EH - 1) + lo, 0, 0)
    return pl.pallas_call(
        _ffn_half_body,
        grid=(EH + 1,),
        in_specs=[
            pl.BlockSpec((CAP, C), lambda e: (e + lo, 0)),
            pl.BlockSpec((1, C, F), wmap),
            pl.BlockSpec((1, 1, F), wmap),
            pl.BlockSpec((1, F, C), wmap),
            pl.BlockSpec((1, 1, C), wmap),
        ],
        out_specs=pl.BlockSpec((CAP, C), lambda e: (e, 0)),
        out_shape=jax.ShapeDtypeStruct((OBH, C), jnp.float32),
        interpret=interpret,
    )(disp, W1, b1.reshape(E, 1, F), W2, b2.reshape(E, 1, C))


CCH = 16
NCH = TPW // CCH


def _remap_half(idx_ref, lo):
    for j in range(TPW // 16):
        sl = pl.ds(j * 16, 16)
        a = idx_ref[sl]
        idx_ref[sl] = jnp.where((a >= lo) & (a < lo + HALF), a - lo, HALF)


def _combine_a_body(ob_hbm, dw1_hbm, dw2_hbm, wv1_hbm, wv2_hbm, out_hbm,
                    i1_v, i2_v, w1_v, w2_v, r1_v, r2_v,
                    g1a, g2a, g1b, g2b, so):
    wid = lax.axis_index("s") * NC + lax.axis_index("c")
    base = wid * TPW
    pltpu.sync_copy(dw1_hbm.at[pl.ds(base, TPW)], i1_v)
    pltpu.sync_copy(dw2_hbm.at[pl.ds(base, TPW)], i2_v)
    pltpu.sync_copy(wv1_hbm.at[pl.ds(base, TPW)], w1_v)
    pltpu.sync_copy(wv2_hbm.at[pl.ds(base, TPW)], w2_v)
    _remap_half(i1_v, 0)
    _remap_half(i2_v, 0)

    def issue(k, s1, s2):
        sl = pl.ds(k * CCH, CCH)
        c1 = pltpu.async_copy(ob_hbm.at[i1_v.at[sl]], r1_v.at[sl], s1)
        c2 = pltpu.async_copy(ob_hbm.at[i2_v.at[sl]], r2_v.at[sl], s2)
        return (c1, c2)

    def compute(k):
        def row_body(i, carry):
            wb1 = w1_v[i, :]
            wb2 = w2_v[i, :]
            for cch in range(C // 16):
                sl = pl.ds(cch * 16, 16)
                r1_v[i, sl] = r1_v[i, sl] * wb1 + r2_v[i, sl] * wb2
            return carry

        lax.fori_loop(k * CCH, (k + 1) * CCH, row_body, 0)

    sems = [(g1a, g2a), (g1b, g2b)]
    inflight = {0: issue(0, *sems[0]), 1: issue(1, *sems[1])}
    stores = []
    for k in range(NCH):
        c1, c2 = inflight.pop(k)
        c1.wait()
        c2.wait()
        compute(k)
        sl = pl.ds(k * CCH, CCH)
        stores.append(pltpu.async_copy(
            r1_v.at[sl], out_hbm.at[pl.ds(base + k * CCH, CCH)], so))
        if k + 2 < NCH:
            inflight[k + 2] = issue(k + 2, *sems[k % 2])
    for st in stores:
        st.wait()


def _combine_b_body(ob_hbm, dw1_hbm, dw2_hbm, wv1_hbm, wv2_hbm, part_hbm,
                    out_hbm,
                    i1_v, i2_v, w1_v, w2_v, r1_v, r2_v, r3_v,
                    g1a, g2a, pa, g1b, g2b, pb, g1c, g2c, pc, so):
    wid = lax.axis_index("s") * NC + lax.axis_index("c")
    base = wid * TPW
    pltpu.sync_copy(dw1_hbm.at[pl.ds(base, TPW)], i1_v)
    pltpu.sync_copy(dw2_hbm.at[pl.ds(base, TPW)], i2_v)
    pltpu.sync_copy(wv1_hbm.at[pl.ds(base, TPW)], w1_v)
    pltpu.sync_copy(wv2_hbm.at[pl.ds(base, TPW)], w2_v)
    _remap_half(i1_v, HALF)
    _remap_half(i2_v, HALF)

    sems = [(g1a, g2a, pa), (g1b, g2b, pb), (g1c, g2c, pc)]

    def issue(k):
        s1, s2, sp = sems[k % 3]
        sl = pl.ds(k * CCH, CCH)
        rsl = pl.ds((k % 3) * CCH, CCH)
        c1 = pltpu.async_copy(ob_hbm.at[i1_v.at[sl]], r1_v.at[rsl], s1)
        c2 = pltpu.async_copy(ob_hbm.at[i2_v.at[sl]], r2_v.at[rsl], s2)
        c3 = pltpu.async_copy(part_hbm.at[pl.ds(base + k * CCH, CCH)],
                              r3_v.at[rsl], sp)
        return (c1, c2, c3)

    def compute(k):
        off = (k % 3) * CCH - k * CCH

        def row_body(i, carry):
            wb1 = w1_v[i, :]
            wb2 = w2_v[i, :]
            r = i + off
            for cch in range(C // 16):
                sl = pl.ds(cch * 16, 16)
                r1_v[r, sl] = (r3_v[r, sl]
                               + r1_v[r, sl] * wb1 + r2_v[r, sl] * wb2)
            return carry

        lax.fori_loop(k * CCH, (k + 1) * CCH, row_body, 0)

    inflight = {0: issue(0), 1: issue(1), 2: issue(2)}
    stores = {}
    for k in range(NCH):
        c1, c2, c3 = inflight.pop(k)
        c1.wait()
        c2.wait()
        c3.wait()
        compute(k)
        rsl = pl.ds((k % 3) * CCH, CCH)
        stores[k] = pltpu.async_copy(
            r1_v.at[rsl], out_hbm.at[pl.ds(base + k * CCH, CCH)], so)
        if k + 3 < NCH:
            stores.pop(k).wait()
            inflight[k + 3] = issue(k + 3)
    for st in stores.values():
        st.wait()


def _run_combine_a(ob, dw1, dw2, wv1, wv2, interpret=False):
    return pl.kernel(
        _combine_a_body,
        out_type=jax.ShapeDtypeStruct((N, C), jnp.float32),
        mesh=_sc_mesh(),
        scratch_types=[
            pltpu.VMEM((TPW,), jnp.int32),
            pltpu.VMEM((TPW,), jnp.int32),
            pltpu.VMEM((TPW, 16), jnp.float32),
            pltpu.VMEM((TPW, 16), jnp.float32),
            pltpu.VMEM((TPW, C), jnp.float32),
            pltpu.VMEM((TPW, C), jnp.float32),
            pltpu.SemaphoreType.DMA,
            pltpu.SemaphoreType.DMA,
            pltpu.SemaphoreType.DMA,
            pltpu.SemaphoreType.DMA,
            pltpu.SemaphoreType.DMA,
        ],
        interpret=interpret,
    )(ob, dw1, dw2, wv1, wv2)


def _run_combine_b(ob, dw1, dw2, wv1, wv2, part, interpret=False):
    return pl.kernel(
        _combine_b_body,
        out_type=jax.ShapeDtypeStruct((N, C), jnp.float32),
        mesh=_sc_mesh(),
        scratch_types=[
            pltpu.VMEM((TPW,), jnp.int32),
            pltpu.VMEM((TPW,), jnp.int32),
            pltpu.VMEM((TPW, 16), jnp.float32),
            pltpu.VMEM((TPW, 16), jnp.float32),
            pltpu.VMEM((3 * CCH, C), jnp.float32),
            pltpu.VMEM((3 * CCH, C), jnp.float32),
            pltpu.VMEM((3 * CCH, C), jnp.float32),
            pltpu.SemaphoreType.DMA,
            pltpu.SemaphoreType.DMA,
            pltpu.SemaphoreType.DMA,
            pltpu.SemaphoreType.DMA,
            pltpu.SemaphoreType.DMA,
            pltpu.SemaphoreType.DMA,
            pltpu.SemaphoreType.DMA,
            pltpu.SemaphoreType.DMA,
            pltpu.SemaphoreType.DMA,
            pltpu.SemaphoreType.DMA,
        ],
        interpret=interpret,
    )(ob, dw1, dw2, wv1, wv2, part)


def kernel(x, Wr, W1, b1, W2, b2):
    xf = x.reshape(N, C)
    dw1, dw2, wv1, wv2 = _run_router(xf, Wr)
    disp = _run_dispatch(xf, dw1, dw2)
    obA = _run_ffn_half(disp, W1, b1, W2, b2, 0)
    obB = _run_ffn_half(disp, W1, b1, W2, b2, EH)
    partA = _run_combine_a(obA, dw1, dw2, wv1, wv2)
    out = _run_combine_b(obB, dw1, dw2, wv1, wv2, partA)
    return out.reshape(B, T, C)


# speedup vs baseline: 1.6797x; 1.6797x over previous
import functools

import jax
import jax.numpy as jnp
from jax import lax
from jax.experimental import pallas as pl
from jax.experimental.pallas import tpu as pltpu
from jax.experimental.pallas import tpu_sc as plsc

B = 1
T = 2048
C = 768
F = 1536
E = 64
K = 2
CAP = 128
N = B * T

NC = 2
NS = 16
NW = NC * NS
TPW = N // NW
TB = 512
NB = N // TB
DISP_ROWS = E * CAP
TRASH0 = DISP_ROWS + TPW
EPB = 1
DISP_PAD = DISP_ROWS + 2 * TPW


def _router_body(x_ref, wr_ref, dw1_ref, dw2_ref, wv1_ref, wv2_ref, cnt_ref):
    i = pl.program_id(0)

    @pl.when(i == 0)
    def _init():
        cnt_ref[...] = jnp.zeros_like(cnt_ref)

    x = x_ref[...]
    logits = jnp.dot(x, wr_ref[...], preferred_element_type=jnp.float32)
    m = jnp.max(logits, axis=-1, keepdims=True)
    p = jnp.exp(logits - m)
    p = p / jnp.sum(p, axis=-1, keepdims=True)

    cols = lax.broadcasted_iota(jnp.int32, (TB, E), 1)
    p1 = jnp.max(p, axis=-1, keepdims=True)
    e1 = jnp.min(jnp.where(p == p1, cols, E), axis=-1, keepdims=True)
    pm = jnp.where(cols == e1, -1.0, p)
    p2 = jnp.max(pm, axis=-1, keepdims=True)
    e2 = jnp.min(jnp.where(pm == p2, cols, E), axis=-1, keepdims=True)

    denom = p1 + p2 + 1e-9
    w1 = p1 / denom
    w2 = p2 / denom

    oh1 = (cols == e1).astype(jnp.float32)
    oh2 = (cols == e2).astype(jnp.float32)
    oh = oh1 + oh2
    r = lax.broadcasted_iota(jnp.int32, (TB, TB), 0)
    ccol = lax.broadcasted_iota(jnp.int32, (TB, TB), 1)
    tril = (r > ccol).astype(jnp.float32)
    carry = cnt_ref[0:1, :]
    cnt_excl = carry + jnp.dot(tril, oh, preferred_element_type=jnp.float32)
    cnt_ref[0:1, :] = carry + jnp.sum(oh, axis=0, keepdims=True)

    pos1 = jnp.sum(cnt_excl * oh1, axis=-1, keepdims=True).astype(jnp.int32)
    pos2 = jnp.sum(cnt_excl * oh2, axis=-1, keepdims=True).astype(jnp.int32)
    v1 = pos1 < CAP
    v2 = pos2 < CAP
    tok = lax.broadcasted_iota(jnp.int32, (TB, 1), 0)
    trash = TRASH0 + (tok % TPW)
    d1 = jnp.where(v1, e1 * CAP + pos1, trash)
    d2 = jnp.where(v2, e2 * CAP + pos2, trash)
    wv1 = jnp.where(v1, w1, 0.0)
    wv2 = jnp.where(v2, w2, 0.0)

    dw1_ref[...] = d1.reshape(1, 1, TB)
    dw2_ref[...] = d2.reshape(1, 1, TB)
    wv1_ref[...] = jnp.broadcast_to(wv1, (TB, 16)).reshape(1, TB, 16)
    wv2_ref[...] = jnp.broadcast_to(wv2, (TB, 16)).reshape(1, TB, 16)


def _run_router(xf, Wr, interpret=False):
    out3 = (
        jax.ShapeDtypeStruct((NB, 1, TB), jnp.int32),
        jax.ShapeDtypeStruct((NB, 1, TB), jnp.int32),
        jax.ShapeDtypeStruct((NB, TB, 16), jnp.float32),
        jax.ShapeDtypeStruct((NB, TB, 16), jnp.float32),
    )
    blk3 = pl.BlockSpec((1, 1, TB), lambda i: (i, 0, 0))
    blkw = pl.BlockSpec((1, TB, 16), lambda i: (i, 0, 0))
    dw1, dw2, wv1, wv2 = pl.pallas_call(
        _router_body,
        grid=(NB,),
        in_specs=[
            pl.BlockSpec((TB, C), lambda i: (i, 0)),
            pl.BlockSpec((C, E), lambda i: (0, 0)),
        ],
        out_specs=(blk3, blk3, blkw, blkw),
        out_shape=out3,
        scratch_shapes=[pltpu.VMEM((8, E), jnp.float32)],
        interpret=interpret,
    )(xf, Wr)
    return (dw1.reshape(N), dw2.reshape(N),
            wv1.reshape(N, 16), wv2.reshape(N, 16))


def _dispatch_body(x_hbm, dw1_hbm, dw2_hbm, disp_hbm, i1_v, i2_v, rows_v,
                   s1, s2):
    wid = lax.axis_index("s") * NC + lax.axis_index("c")
    base = wid * TPW
    pltpu.sync_copy(dw1_hbm.at[pl.ds(base, TPW)], i1_v)
    pltpu.sync_copy(dw2_hbm.at[pl.ds(base, TPW)], i2_v)
    pltpu.sync_copy(x_hbm.at[pl.ds(base, TPW)], rows_v)
    cp1 = pltpu.async_copy(rows_v, disp_hbm.at[i1_v], s1)
    cp2 = pltpu.async_copy(rows_v, disp_hbm.at[i2_v], s2)
    cp1.wait()
    cp2.wait()


def _sc_mesh():
    return plsc.VectorSubcoreMesh(core_axis_name="c", subcore_axis_name="s",
                                  num_cores=NC, num_subcores=NS)


def _run_dispatch(xf, dw1, dw2, interpret=False):
    mesh = _sc_mesh()
    return pl.kernel(
        _dispatch_body,
        out_type=jax.ShapeDtypeStruct((DISP_PAD, C), jnp.float32),
        mesh=mesh,
        scratch_types=[
            pltpu.VMEM((TPW,), jnp.int32),
            pltpu.VMEM((TPW,), jnp.int32),
            pltpu.VMEM((TPW, C), jnp.float32),
            pltpu.SemaphoreType.DMA,
            pltpu.SemaphoreType.DMA,
        ],
        interpret=interpret,
    )(xf, dw1, dw2)


EH = E // 2
HALF = EH * CAP
OBH = HALF + CAP


def _ffn_half_body(x_ref, w1_ref, b1_ref, w2_ref, b2_ref, o_ref):
    e = pl.program_id(0)

    @pl.when(e < EH)
    def _():
        x = x_ref[...]
        h = jnp.dot(x, w1_ref[0], preferred_element_type=jnp.float32)
        h = jnp.maximum(h + b1_ref[0], 0.0)
        o = jnp.dot(h, w2_ref[0], preferred_element_type=jnp.float32)
        o_ref[...] = o + b2_ref[0]

    @pl.when(e == EH)
    def _():
        o_ref[...] = jnp.zeros((CAP, C), jnp.float32)


def _run_ffn_half(disp, W1, b1, W2, b2, lo, interpret=False):
    wmap = lambda e: (jnp.minimum(e, EH - 1) + lo, 0, 0)
    return pl.pallas_call(
        _ffn_half_body,
        grid=(EH + 1,),
        in_specs=[
            pl.BlockSpec((CAP, C), lambda e: (e + lo, 0)),
            pl.BlockSpec((1, C, F), wmap),
            pl.BlockSpec((1, 1, F), wmap),
            pl.BlockSpec((1, F, C), wmap),
            pl.BlockSpec((1, 1, C), wmap),
        ],
        out_specs=pl.BlockSpec((CAP, C), lambda e: (e, 0)),
        out_shape=jax.ShapeDtypeStruct((OBH, C), jnp.float32),
        interpret=interpret,
    )(disp, W1, b1.reshape(E, 1, F), W2, b2.reshape(E, 1, C))


CCH = 16
NCH = TPW // CCH


def _remap_half(idx_ref, lo, base):
    for j in range(TPW // 16):
        sl = pl.ds(j * 16, 16)
        a = idx_ref[sl]
        z = HALF + ((base + j * 16 + lax.iota(jnp.int32, 16)) & (CAP - 1))
        idx_ref[sl] = jnp.where((a >= lo) & (a < lo + HALF), a - lo, z)


def _combine_a_body(ob_hbm, dw1_hbm, dw2_hbm, wv1_hbm, wv2_hbm, out_hbm,
                    i1_v, i2_v, w1_v, w2_v, r1_v, r2_v,
                    g1a, g2a, g1b, g2b, so):
    wid = lax.axis_index("s") * NC + lax.axis_index("c")
    base = wid * TPW
    pltpu.sync_copy(dw1_hbm.at[pl.ds(base, TPW)], i1_v)
    pltpu.sync_copy(dw2_hbm.at[pl.ds(base, TPW)], i2_v)
    pltpu.sync_copy(wv1_hbm.at[pl.ds(base, TPW)], w1_v)
    pltpu.sync_copy(wv2_hbm.at[pl.ds(base, TPW)], w2_v)
    _remap_half(i1_v, 0, base)
    _remap_half(i2_v, 0, base)

    def issue(k, s1, s2):
        sl = pl.ds(k * CCH, CCH)
        c1 = pltpu.async_copy(ob_hbm.at[i1_v.at[sl]], r1_v.at[sl], s1)
        c2 = pltpu.async_copy(ob_hbm.at[i2_v.at[sl]], r2_v.at[sl], s2)
        return (c1, c2)

    def compute(k):
        def row_body(i, carry):
            wb1 = w1_v[i, :]
            wb2 = w2_v[i, :]
            for cch in range(C // 16):
                sl = pl.ds(cch * 16, 16)
                r1_v[i, sl] = r1_v[i, sl] * wb1 + r2_v[i, sl] * wb2
            return carry

        lax.fori_loop(k * CCH, (k + 1) * CCH, row_body, 0)

    sems = [(g1a, g2a), (g1b, g2b)]
    inflight = {0: issue(0, *sems[0]), 1: issue(1, *sems[1])}
    stores = []
    for k in range(NCH):
        c1, c2 = inflight.pop(k)
        c1.wait()
        c2.wait()
        compute(k)
        sl = pl.ds(k * CCH, CCH)
        stores.append(pltpu.async_copy(
            r1_v.at[sl], out_hbm.at[pl.ds(base + k * CCH, CCH)], so))
        if k + 2 < NCH:
            inflight[k + 2] = issue(k + 2, *sems[k % 2])
    for st in stores:
        st.wait()


def _combine_b_body(ob_hbm, dw1_hbm, dw2_hbm, wv1_hbm, wv2_hbm, part_hbm,
                    out_hbm,
                    i1_v, i2_v, w1_v, w2_v, r1_v, r2_v, r3_v,
                    g1a, g2a, pa, g1b, g2b, pb, g1c, g2c, pc, so):
    wid = lax.axis_index("s") * NC + lax.axis_index("c")
    base = wid * TPW
    pltpu.sync_copy(dw1_hbm.at[pl.ds(base, TPW)], i1_v)
    pltpu.sync_copy(dw2_hbm.at[pl.ds(base, TPW)], i2_v)
    pltpu.sync_copy(wv1_hbm.at[pl.ds(base, TPW)], w1_v)
    pltpu.sync_copy(wv2_hbm.at[pl.ds(base, TPW)], w2_v)
    _remap_half(i1_v, HALF, base)
    _remap_half(i2_v, HALF, base)

    sems = [(g1a, g2a, pa), (g1b, g2b, pb), (g1c, g2c, pc)]

    def issue(k):
        s1, s2, sp = sems[k % 3]
        sl = pl.ds(k * CCH, CCH)
        rsl = pl.ds((k % 3) * CCH, CCH)
        c1 = pltpu.async_copy(ob_hbm.at[i1_v.at[sl]], r1_v.at[rsl], s1)
        c2 = pltpu.async_copy(ob_hbm.at[i2_v.at[sl]], r2_v.at[rsl], s2)
        c3 = pltpu.async_copy(part_hbm.at[pl.ds(base + k * CCH, CCH)],
                              r3_v.at[rsl], sp)
        return (c1, c2, c3)

    def compute(k):
        off = (k % 3) * CCH - k * CCH

        def row_body(i, carry):
            wb1 = w1_v[i, :]
            wb2 = w2_v[i, :]
            r = i + off
            for cch in range(C // 16):
                sl = pl.ds(cch * 16, 16)
                r1_v[r, sl] = (r3_v[r, sl]
                               + r1_v[r, sl] * wb1 + r2_v[r, sl] * wb2)
            return carry

        lax.fori_loop(k * CCH, (k + 1) * CCH, row_body, 0)

    inflight = {0: issue(0), 1: issue(1), 2: issue(2)}
    stores = {}
    for k in range(NCH):
        c1, c2, c3 = inflight.pop(k)
        c1.wait()
        c2.wait()
        c3.wait()
        compute(k)
        rsl = pl.ds((k % 3) * CCH, CCH)
        stores[k] = pltpu.async_copy(
            r1_v.at[rsl], out_hbm.at[pl.ds(base + k * CCH, CCH)], so)
        if k + 3 < NCH:
            stores.pop(k).wait()
            inflight[k + 3] = issue(k + 3)
    for st in stores.values():
        st.wait()


def _run_combine_a(ob, dw1, dw2, wv1, wv2, interpret=False):
    return pl.kernel(
        _combine_a_body,
        out_type=jax.ShapeDtypeStruct((N, C), jnp.float32),
        mesh=_sc_mesh(),
        scratch_types=[
            pltpu.VMEM((TPW,), jnp.int32),
            pltpu.VMEM((TPW,), jnp.int32),
            pltpu.VMEM((TPW, 16), jnp.float32),
            pltpu.VMEM((TPW, 16), jnp.float32),
            pltpu.VMEM((TPW, C), jnp.float32),
            pltpu.VMEM((TPW, C), jnp.float32),
            pltpu.SemaphoreType.DMA,
            pltpu.SemaphoreType.DMA,
            pltpu.SemaphoreType.DMA,
            pltpu.SemaphoreType.DMA,
            pltpu.SemaphoreType.DMA,
        ],
        interpret=interpret,
    )(ob, dw1, dw2, wv1, wv2)


def _run_combine_b(ob, dw1, dw2, wv1, wv2, part, interpret=False):
    return pl.kernel(
        _combine_b_body,
        out_type=jax.ShapeDtypeStruct((N, C), jnp.float32),
        mesh=_sc_mesh(),
        scratch_types=[
            pltpu.VMEM((TPW,), jnp.int32),
            pltpu.VMEM((TPW,), jnp.int32),
            pltpu.VMEM((TPW, 16), jnp.float32),
            pltpu.VMEM((TPW, 16), jnp.float32),
            pltpu.VMEM((3 * CCH, C), jnp.float32),
            pltpu.VMEM((3 * CCH, C), jnp.float32),
            pltpu.VMEM((3 * CCH, C), jnp.float32),
            pltpu.SemaphoreType.DMA,
            pltpu.SemaphoreType.DMA,
            pltpu.SemaphoreType.DMA,
            pltpu.SemaphoreType.DMA,
            pltpu.SemaphoreType.DMA,
            pltpu.SemaphoreType.DMA,
            pltpu.SemaphoreType.DMA,
            pltpu.SemaphoreType.DMA,
            pltpu.SemaphoreType.DMA,
            pltpu.SemaphoreType.DMA,
        ],
        interpret=interpret,
    )(ob, dw1, dw2, wv1, wv2, part)


def kernel(x, Wr, W1, b1, W2, b2):
    xf = x.reshape(N, C)
    dw1, dw2, wv1, wv2 = _run_router(xf, Wr)
    disp = _run_dispatch(xf, dw1, dw2)
    obA = _run_ffn_half(disp, W1, b1, W2, b2, 0)
    obB = _run_ffn_half(disp, W1, b1, W2, b2, EH)
    partA = _run_combine_a(obA, dw1, dw2, wv1, wv2)
    out = _run_combine_b(obB, dw1, dw2, wv1, wv2, partA)
    return out.reshape(B, T, C)


# speedup vs baseline: 1.8074x; 1.0760x over previous
import functools

import jax
import jax.numpy as jnp
from jax import lax
from jax.experimental import pallas as pl
from jax.experimental.pallas import tpu as pltpu
from jax.experimental.pallas import tpu_sc as plsc

B = 1
T = 2048
C = 768
F = 1536
E = 64
K = 2
CAP = 128
N = B * T

NC = 2
NS = 16
NW = NC * NS
TPW = N // NW
TB = 512
NB = N // TB
DISP_ROWS = E * CAP
TRASH0 = DISP_ROWS + TPW
EPB = 1
DISP_PAD = DISP_ROWS + 2 * TPW


def _router_body(x_ref, wr_ref, dw1_ref, dw2_ref, wv1_ref, wv2_ref, cnt_ref):
    i = pl.program_id(0)

    @pl.when(i == 0)
    def _init():
        cnt_ref[...] = jnp.zeros_like(cnt_ref)

    x = x_ref[...]
    logits = jnp.dot(x, wr_ref[...], preferred_element_type=jnp.float32)
    m = jnp.max(logits, axis=-1, keepdims=True)
    p = jnp.exp(logits - m)
    p = p / jnp.sum(p, axis=-1, keepdims=True)

    cols = lax.broadcasted_iota(jnp.int32, (TB, E), 1)
    p1 = jnp.max(p, axis=-1, keepdims=True)
    e1 = jnp.min(jnp.where(p == p1, cols, E), axis=-1, keepdims=True)
    pm = jnp.where(cols == e1, -1.0, p)
    p2 = jnp.max(pm, axis=-1, keepdims=True)
    e2 = jnp.min(jnp.where(pm == p2, cols, E), axis=-1, keepdims=True)

    denom = p1 + p2 + 1e-9
    w1 = p1 / denom
    w2 = p2 / denom

    oh1 = (cols == e1).astype(jnp.float32)
    oh2 = (cols == e2).astype(jnp.float32)
    oh = oh1 + oh2
    r = lax.broadcasted_iota(jnp.int32, (TB, TB), 0)
    ccol = lax.broadcasted_iota(jnp.int32, (TB, TB), 1)
    tril = (r > ccol).astype(jnp.float32)
    carry = cnt_ref[0:1, :]
    cnt_excl = carry + jnp.dot(tril, oh, preferred_element_type=jnp.float32)
    cnt_ref[0:1, :] = carry + jnp.sum(oh, axis=0, keepdims=True)

    pos1 = jnp.sum(cnt_excl * oh1, axis=-1, keepdims=True).astype(jnp.int32)
    pos2 = jnp.sum(cnt_excl * oh2, axis=-1, keepdims=True).astype(jnp.int32)
    v1 = pos1 < CAP
    v2 = pos2 < CAP
    tok = lax.broadcasted_iota(jnp.int32, (TB, 1), 0)
    trash = TRASH0 + (tok % TPW)
    d1 = jnp.where(v1, e1 * CAP + pos1, trash)
    d2 = jnp.where(v2, e2 * CAP + pos2, trash)
    wv1 = jnp.where(v1, w1, 0.0)
    wv2 = jnp.where(v2, w2, 0.0)

    dw1_ref[...] = d1.reshape(1, 1, TB)
    dw2_ref[...] = d2.reshape(1, 1, TB)
    wv1_ref[...] = jnp.broadcast_to(wv1, (TB, 16)).reshape(1, TB, 16)
    wv2_ref[...] = jnp.broadcast_to(wv2, (TB, 16)).reshape(1, TB, 16)


def _run_router(xf, Wr, interpret=False):
    out3 = (
        jax.ShapeDtypeStruct((NB, 1, TB), jnp.int32),
        jax.ShapeDtypeStruct((NB, 1, TB), jnp.int32),
        jax.ShapeDtypeStruct((NB, TB, 16), jnp.float32),
        jax.ShapeDtypeStruct((NB, TB, 16), jnp.float32),
    )
    blk3 = pl.BlockSpec((1, 1, TB), lambda i: (i, 0, 0))
    blkw = pl.BlockSpec((1, TB, 16), lambda i: (i, 0, 0))
    dw1, dw2, wv1, wv2 = pl.pallas_call(
        _router_body,
        grid=(NB,),
        in_specs=[
            pl.BlockSpec((TB, C), lambda i: (i, 0)),
            pl.BlockSpec((C, E), lambda i: (0, 0)),
        ],
        out_specs=(blk3, blk3, blkw, blkw),
        out_shape=out3,
        scratch_shapes=[pltpu.VMEM((8, E), jnp.float32)],
        interpret=interpret,
    )(xf, Wr)
    return (dw1.reshape(N), dw2.reshape(N),
            wv1.reshape(N, 16), wv2.reshape(N, 16))


def _dispatch_body(x_hbm, dw1_hbm, dw2_hbm, disp_hbm, i1_v, i2_v, rows_v,
                   s1, s2):
    wid = lax.axis_index("s") * NC + lax.axis_index("c")
    base = wid * TPW
    pltpu.sync_copy(dw1_hbm.at[pl.ds(base, TPW)], i1_v)
    pltpu.sync_copy(dw2_hbm.at[pl.ds(base, TPW)], i2_v)
    pltpu.sync_copy(x_hbm.at[pl.ds(base, TPW)], rows_v)
    cp1 = pltpu.async_copy(rows_v, disp_hbm.at[i1_v], s1)
    cp2 = pltpu.async_copy(rows_v, disp_hbm.at[i2_v], s2)
    cp1.wait()
    cp2.wait()


def _sc_mesh():
    return plsc.VectorSubcoreMesh(core_axis_name="c", subcore_axis_name="s",
                                  num_cores=NC, num_subcores=NS)


def _run_dispatch(xf, dw1, dw2, interpret=False):
    mesh = _sc_mesh()
    return pl.kernel(
        _dispatch_body,
        out_type=jax.ShapeDtypeStruct((DISP_PAD, C), jnp.float32),
        mesh=mesh,
        scratch_types=[
            pltpu.VMEM((TPW,), jnp.int32),
            pltpu.VMEM((TPW,), jnp.int32),
            pltpu.VMEM((TPW, C), jnp.float32),
            pltpu.SemaphoreType.DMA,
            pltpu.SemaphoreType.DMA,
        ],
        interpret=interpret,
    )(xf, dw1, dw2)


def _ffn_body(x_ref, w1_ref, b1_ref, w2_ref, b2_ref, o_ref):
    x = x_ref[...]
    h = jnp.dot(x, w1_ref[0], preferred_element_type=jnp.float32)
    h = jnp.maximum(h + b1_ref[0], 0.0)
    o = jnp.dot(h, w2_ref[0], preferred_element_type=jnp.float32)
    o_ref[...] = o + b2_ref[0]


def _run_ffn(disp, W1, b1, W2, b2, interpret=False):
    return pl.pallas_call(
        _ffn_body,
        grid=(E,),
        in_specs=[
            pl.BlockSpec((CAP, C), lambda e: (e, 0)),
            pl.BlockSpec((1, C, F), lambda e: (e, 0, 0)),
            pl.BlockSpec((1, 1, F), lambda e: (e, 0, 0)),
            pl.BlockSpec((1, F, C), lambda e: (e, 0, 0)),
            pl.BlockSpec((1, 1, C), lambda e: (e, 0, 0)),
        ],
        out_specs=pl.BlockSpec((CAP, C), lambda e: (e, 0)),
        out_shape=jax.ShapeDtypeStruct((DISP_ROWS, C), jnp.float32),
        interpret=interpret,
    )(disp, W1, b1.reshape(E, 1, F), W2, b2.reshape(E, 1, C))


CCH = 16
NCH = TPW // CCH


def _combine_body(ob_hbm, dw1_hbm, dw2_hbm, wv1_hbm, wv2_hbm, out_hbm,
                  i1_v, i2_v, w1_v, w2_v, r1_v, r2_v,
                  g1a, g2a, g1b, g2b, so):
    wid = lax.axis_index("s") * NC + lax.axis_index("c")
    base = wid * TPW
    pltpu.sync_copy(dw1_hbm.at[pl.ds(base, TPW)], i1_v)
    pltpu.sync_copy(dw2_hbm.at[pl.ds(base, TPW)], i2_v)
    pltpu.sync_copy(wv1_hbm.at[pl.ds(base, TPW)], w1_v)
    pltpu.sync_copy(wv2_hbm.at[pl.ds(base, TPW)], w2_v)
    for j in range(TPW // 16):
        sl = pl.ds(j * 16, 16)
        z = (base + j * 16 + lax.iota(jnp.int32, 16)) & (DISP_ROWS - 1)
        a = i1_v[sl]
        i1_v[sl] = jnp.where(a >= DISP_ROWS, z, a)
        b = i2_v[sl]
        i2_v[sl] = jnp.where(b >= DISP_ROWS, z, b)

    def issue(k, s1, s2):
        sl = pl.ds(k * CCH, CCH)
        c1 = pltpu.async_copy(ob_hbm.at[i1_v.at[sl]], r1_v.at[sl], s1)
        c2 = pltpu.async_copy(ob_hbm.at[i2_v.at[sl]], r2_v.at[sl], s2)
        return (c1, c2)

    def compute(k):
        def row_body(i, carry):
            wb1 = w1_v[i, :]
            wb2 = w2_v[i, :]
            m1 = wb1 > 0.0
            m2 = wb2 > 0.0
            for cch in range(C // 16):
                sl = pl.ds(cch * 16, 16)
                a = r1_v[i, sl]
                b = r2_v[i, sl]
                r1_v[i, sl] = (jnp.where(m1, a * wb1, 0.0)
                               + jnp.where(m2, b * wb2, 0.0))
            return carry

        lax.fori_loop(k * CCH, (k + 1) * CCH, row_body, 0)

    sems = [(g1a, g2a), (g1b, g2b)]
    inflight = {0: issue(0, *sems[0]), 1: issue(1, *sems[1])}
    stores = []
    for k in range(NCH):
        c1, c2 = inflight.pop(k)
        c1.wait()
        c2.wait()
        compute(k)
        sl = pl.ds(k * CCH, CCH)
        stores.append(pltpu.async_copy(
            r1_v.at[sl], out_hbm.at[pl.ds(base + k * CCH, CCH)], so))
        if k + 2 < NCH:
            inflight[k + 2] = issue(k + 2, *sems[k % 2])
    for st in stores:
        st.wait()


def _run_combine(ob, dw1, dw2, wv1, wv2, interpret=False):
    return pl.kernel(
        _combine_body,
        out_type=jax.ShapeDtypeStruct((N, C), jnp.float32),
        mesh=_sc_mesh(),
        scratch_types=[
            pltpu.VMEM((TPW,), jnp.int32),
            pltpu.VMEM((TPW,), jnp.int32),
            pltpu.VMEM((TPW, 16), jnp.float32),
            pltpu.VMEM((TPW, 16), jnp.float32),
            pltpu.VMEM((TPW, C), jnp.float32),
            pltpu.VMEM((TPW, C), jnp.float32),
            pltpu.SemaphoreType.DMA,
            pltpu.SemaphoreType.DMA,
            pltpu.SemaphoreType.DMA,
            pltpu.SemaphoreType.DMA,
            pltpu.SemaphoreType.DMA,
        ],
        interpret=interpret,
    )(ob, dw1, dw2, wv1, wv2)


def kernel(x, Wr, W1, b1, W2, b2):
    xf = x.reshape(N, C)
    dw1, dw2, wv1, wv2 = _run_router(xf, Wr)
    disp = _run_dispatch(xf, dw1, dw2)
    ob = _run_ffn(disp, W1, b1, W2, b2)
    out = _run_combine(ob, dw1, dw2, wv1, wv2)
    return out.reshape(B, T, C)
